# Initial kernel scaffold; baseline (speedup 1.0000x reference)
import jax
import jax.numpy as jnp
from jax.experimental import pallas as pl


def kernel(x, Wqkv, bqkv, Wout, bout, group_scale, anchor_idx):
    raise NotImplementedError("write your pallas kernel here")


# speedup vs baseline: 15.7954x; 15.7954x over previous
import functools

import jax
import jax.numpy as jnp
from jax.experimental import pallas as pl
from jax.experimental.pallas import tpu as pltpu

_H = 12
_HALO = 10
_OFFS = (-3, -2, -1, 1, 2, 3, -10, -5, 5, 10)
_GROUP = (0, 0, 0, 0, 0, 0, 1, 1, 1, 1, 2, 2)


def _fused_kernel(xp_ref, wqT_ref, wkvT_ref, bq_ref, bkv_ref, woutT_ref,
                  bout_ref, lw_ref, out_ref, *, blk, S, D, dh):
    i = pl.program_id(1)
    W = blk + 2 * _HALO
    f32 = jnp.float32

    xh = xp_ref[0, pl.ds(i * blk, W), :]
    x0 = xp_ref[0, pl.ds(_HALO, 1), :]
    xS = xp_ref[0, pl.ds(S + _HALO - 1, 1), :]
    xkv = jnp.concatenate([xh, x0, xS], axis=0)

    q = (jnp.dot(xh[_HALO:_HALO + blk], wqT_ref[...],
                 preferred_element_type=f32) + bq_ref[...])
    kv = (jnp.dot(xkv, wkvT_ref[...], preferred_element_type=f32)
          + bkv_ref[...])
    kh, vh = kv[:, :D], kv[:, D:]

    scale = dh ** -0.5
    rows = jax.lax.broadcasted_iota(jnp.int32, (D, _H), 0)
    cols = jax.lax.broadcasted_iota(jnp.int32, (D, _H), 1)
    Ms = jnp.where(rows // dh == cols, scale, 0.0).astype(f32)
    rows_e = jax.lax.broadcasted_iota(jnp.int32, (_H, D), 0)
    cols_e = jax.lax.broadcasted_iota(jnp.int32, (_H, D), 1)
    E = jnp.where(cols_e // dh == rows_e, 1.0, 0.0).astype(f32)

    ks = [kh[_HALO + o:_HALO + o + blk] for o in _OFFS]
    ks += [kh[W:W + 1], kh[W + 1:W + 2]]
    vs = [vh[_HALO + o:_HALO + o + blk] for o in _OFFS]
    vs += [vh[W:W + 1], vh[W + 1:W + 2]]

    Ls = [jnp.dot(q * ks[a], Ms, preferred_element_type=f32) + lw_ref[a]
          for a in range(12)]
    m = functools.reduce(jnp.maximum, Ls)
    acc = jnp.zeros((blk, D), f32)
    Z = jnp.zeros((blk, _H), f32)
    for a in range(12):
        e = jnp.exp(Ls[a] - m)
        Z = Z + e
        acc = acc + jnp.dot(e, E, preferred_element_type=f32) * vs[a]
    attn_out = acc / jnp.dot(Z, E, preferred_element_type=f32)

    out_ref[0] = (jnp.dot(attn_out, woutT_ref[...],
                          preferred_element_type=f32) + bout_ref[...])


def kernel(x, Wqkv, bqkv, Wout, bout, group_scale, anchor_idx):
    B, S, D = x.shape
    dh = D // _H
    blk = 512
    nb = S // blk

    x_pad = jnp.concatenate([
        jnp.broadcast_to(x[:, :1], (B, _HALO, D)), x,
        jnp.broadcast_to(x[:, -1:], (B, _HALO, D))], axis=1)
    WqT = Wqkv[:D].T
    WkvT = Wqkv[D:].T
    bq = bqkv[:D].reshape(1, D)
    bkv = bqkv[D:].reshape(1, 2 * D)
    WoutT = Wout.T
    bout2 = bout.reshape(1, D)
    lw = jnp.log(jax.nn.softmax(group_scale))[jnp.array(_GROUP)]

    grid = (B, nb)
    return pl.pallas_call(
        functools.partial(_fused_kernel, blk=blk, S=S, D=D, dh=dh),
        grid=grid,
        in_specs=[
            pl.BlockSpec((1, S + 2 * _HALO, D), lambda b, i: (b, 0, 0)),
            pl.BlockSpec((D, D), lambda b, i: (0, 0)),
            pl.BlockSpec((D, 2 * D), lambda b, i: (0, 0)),
            pl.BlockSpec((1, D), lambda b, i: (0, 0)),
            pl.BlockSpec((1, 2 * D), lambda b, i: (0, 0)),
            pl.BlockSpec((D, D), lambda b, i: (0, 0)),
            pl.BlockSpec((1, D), lambda b, i: (0, 0)),
            pl.BlockSpec(memory_space=pltpu.SMEM),
        ],
        out_specs=pl.BlockSpec((1, blk, D), lambda b, i: (b, i, 0)),
        out_shape=jax.ShapeDtypeStruct((B, S, D), jnp.float32),
    )(x_pad, WqT, WkvT, bq, bkv, WoutT, bout2, lw)
